# baseline (speedup 1.0000x reference)
import functools

import jax
import jax.numpy as jnp
from jax.experimental import pallas as pl

_K = 9


def _lorentz_post(y, s):
    col = jax.lax.broadcasted_iota(jnp.int32, y.shape, 1)
    first = y[:, 0:1]
    t = jax.nn.sigmoid(first) * jnp.exp(s) + 1.1
    yn = jnp.where(col == 0, 0.0, y)
    ss = jnp.sum(yn * yn, axis=1, keepdims=True)
    scale = jnp.sqrt((t * t - 1.0) / jnp.maximum(ss, 1e-8))
    return jnp.where(col == 0, t, y * scale)


def _ll_block(x, W, b, s):
    g = jax.nn.gelu(x)
    y = jax.lax.dot_general(g, W, (((1,), (1,)), ((), ())),
                            preferred_element_type=jnp.float32) + b
    return _lorentz_post(y, s)


def _ffn_kernel(xcn_ref, w1_ref, b1_ref, s1_ref, w2_ref, b2_ref, s2_ref,
                o_ref):
    x = jnp.transpose(xcn_ref[0], (1, 0))
    h = _ll_block(x, w1_ref[...], b1_ref[...], s1_ref[0, 0])
    h = _ll_block(h, w2_ref[...], b2_ref[...], s2_ref[0, 0])
    o_ref[0] = h + x


def _graph_kernel(y1v_ref, scu_ref,
                  wg1_ref, wg2_ref, bg_ref, sg_ref,
                  w2a_ref, b2a_ref, s2a_ref, w2b_ref, b2b_ref, s2b_ref,
                  o_ref):
    C = wg1_ref.shape[0]
    f = jnp.transpose(y1v_ref[0], (1, 0))
    N = f.shape[0]
    ones_col = jnp.ones((N, 1), jnp.float32)
    zeros_pad = jnp.zeros((N, 128 - C - 1), jnp.float32)
    f_aug = jnp.concatenate([f, ones_col, zeros_pad], axis=1)

    col = jax.lax.broadcasted_iota(jnp.int32, f_aug.shape, 1)
    f_signed = jnp.where(col == 0, -f_aug,
                         jnp.where(col < C, f_aug, 0.0))
    inner = jax.lax.dot_general(f_signed, f_aug, (((1,), (1,)), ((), ())),
                                preferred_element_type=jnp.float32)

    f_hi = f_aug.astype(jnp.bfloat16)
    f_lo = (f_aug - f_hi.astype(jnp.float32)).astype(jnp.bfloat16)

    neg_inf = jnp.float32(-jnp.inf)
    nbmax = None
    cur = inner
    for _ in range(_K):
        m = jnp.max(cur, axis=1, keepdims=True)
        hit = cur == m
        sel = hit.astype(jnp.bfloat16)
        g = (jax.lax.dot_general(sel, f_hi, (((1,), (0,)), ((), ())),
                                 preferred_element_type=jnp.float32)
             + jax.lax.dot_general(sel, f_lo, (((1,), (0,)), ((), ())),
                                   preferred_element_type=jnp.float32))
        g = g[:, :C] / g[:, C:C + 1]
        nbmax = g if nbmax is None else jnp.maximum(nbmax, g)
        cur = jnp.where(hit, neg_inf, cur)

    rel = nbmax - f
    y = (jax.lax.dot_general(jax.nn.gelu(f), wg1_ref[...],
                             (((1,), (1,)), ((), ())),
                             preferred_element_type=jnp.float32)
         + jax.lax.dot_general(jax.nn.gelu(rel), wg2_ref[...],
                               (((1,), (1,)), ((), ())),
                               preferred_element_type=jnp.float32)
         + bg_ref[...])
    out = _lorentz_post(y, sg_ref[0, 0])

    h = _ll_block(out, w2a_ref[...], b2a_ref[...], s2a_ref[0, 0])
    h = _ll_block(h, w2b_ref[...], b2b_ref[...], s2b_ref[0, 0])
    o_ref[0] = h + out + scu_ref[0]


@functools.partial(jax.jit, static_argnames=())
def kernel(x, W1a, b1a, s1a, W1b, b1b, s1b, Wg, bg, sg, W2a, b2a, s2a,
           W2b, b2b, s2b):
    B, C, H, W = x.shape
    N = H * W
    f32 = jnp.float32

    def v(a):
        return jnp.asarray(a, f32).reshape(1, -1)

    def sc(a):
        return jnp.asarray(a, f32).reshape(1, 1)

    xcn = x.reshape(B, C, N)

    wspec = pl.BlockSpec((C, C), lambda b: (0, 0))
    bspec = pl.BlockSpec((1, C), lambda b: (0, 0))
    sspec = pl.BlockSpec((1, 1), lambda b: (0, 0))

    ffn1 = pl.pallas_call(
        _ffn_kernel,
        grid=(B,),
        in_specs=[
            pl.BlockSpec((1, C, N), lambda b: (b, 0, 0)),
            wspec, bspec, sspec, wspec, bspec, sspec,
        ],
        out_specs=pl.BlockSpec((1, N, C), lambda b: (b, 0, 0)),
        out_shape=jax.ShapeDtypeStruct((B, N, C), f32),
    )
    y1 = ffn1(xcn, W1a, v(b1a), sc(s1a), W1b, v(b1b), sc(s1b))

    y1v = y1.reshape(B, C, N)
    scu = x.reshape(B, N, C)

    Wg1 = Wg[:, :C]
    Wg2 = Wg[:, C:]

    graph = pl.pallas_call(
        _graph_kernel,
        grid=(B,),
        in_specs=[
            pl.BlockSpec((1, C, N), lambda b: (b, 0, 0)),
            pl.BlockSpec((1, N, C), lambda b: (b, 0, 0)),
            wspec, wspec, bspec, sspec,
            wspec, bspec, sspec, wspec, bspec, sspec,
        ],
        out_specs=pl.BlockSpec((1, N, C), lambda b: (b, 0, 0)),
        out_shape=jax.ShapeDtypeStruct((B, N, C), f32),
    )
    z = graph(y1v, scu, Wg1, Wg2, v(bg), sc(sg),
              W2a, v(b2a), sc(s2a), W2b, v(b2b), sc(s2b))

    return z.reshape(B, C, H, W)


# speedup vs baseline: 1.2389x; 1.2389x over previous
import functools

import jax
import jax.numpy as jnp
from jax.experimental import pallas as pl

_K = 9


def _lorentz_post(y, s):
    col = jax.lax.broadcasted_iota(jnp.int32, y.shape, 1)
    first = y[:, 0:1]
    t = jax.nn.sigmoid(first) * jnp.exp(s) + 1.1
    yn = jnp.where(col == 0, 0.0, y)
    ss = jnp.sum(yn * yn, axis=1, keepdims=True)
    scale = jnp.sqrt((t * t - 1.0) / jnp.maximum(ss, 1e-8))
    return jnp.where(col == 0, t, y * scale)


def _ll_block(x, W, b, s):
    g = jax.nn.gelu(x)
    y = jax.lax.dot_general(g, W, (((1,), (1,)), ((), ())),
                            preferred_element_type=jnp.float32) + b
    return _lorentz_post(y, s)


def _ffn_kernel(xcn_ref, w1_ref, b1_ref, s1_ref, w2_ref, b2_ref, s2_ref,
                o_ref):
    x = jnp.transpose(xcn_ref[0], (1, 0))
    h = _ll_block(x, w1_ref[...], b1_ref[...], s1_ref[0, 0])
    h = _ll_block(h, w2_ref[...], b2_ref[...], s2_ref[0, 0])
    o_ref[0] = h + x


def _graph_kernel(y1v_ref, scu_ref,
                  wg1_ref, wg2_ref, bg_ref, sg_ref,
                  w2a_ref, b2a_ref, s2a_ref, w2b_ref, b2b_ref, s2b_ref,
                  o_ref):
    C = wg1_ref.shape[0]
    f = jnp.transpose(y1v_ref[0], (1, 0))
    N = f.shape[0]
    ones_col = jnp.ones((N, 1), jnp.float32)
    zeros_pad = jnp.zeros((N, 128 - C - 1), jnp.float32)
    f_aug = jnp.concatenate([f, ones_col, zeros_pad], axis=1)

    col = jax.lax.broadcasted_iota(jnp.int32, f_aug.shape, 1)
    f_signed = jnp.where(col == 0, -f_aug,
                         jnp.where(col < C, f_aug, 0.0))
    inner = jax.lax.dot_general(f_signed, f_aug, (((1,), (1,)), ((), ())),
                                preferred_element_type=jnp.float32)

    neg_inf = jnp.float32(-jnp.inf)
    nbmax = None
    m_prev = None
    for _ in range(_K):
        if m_prev is None:
            m = jnp.max(inner, axis=1, keepdims=True)
        else:
            m = jnp.max(jnp.where(inner < m_prev, inner, neg_inf),
                        axis=1, keepdims=True)
        sel = (inner == m).astype(jnp.float32)
        g = jax.lax.dot_general(sel, f_aug, (((1,), (0,)), ((), ())),
                                preferred_element_type=jnp.float32)
        g = g[:, :C] / g[:, C:C + 1]
        nbmax = g if nbmax is None else jnp.maximum(nbmax, g)
        m_prev = m

    rel = nbmax - f
    y = (jax.lax.dot_general(jax.nn.gelu(f), wg1_ref[...],
                             (((1,), (1,)), ((), ())),
                             preferred_element_type=jnp.float32)
         + jax.lax.dot_general(jax.nn.gelu(rel), wg2_ref[...],
                               (((1,), (1,)), ((), ())),
                               preferred_element_type=jnp.float32)
         + bg_ref[...])
    out = _lorentz_post(y, sg_ref[0, 0])

    h = _ll_block(out, w2a_ref[...], b2a_ref[...], s2a_ref[0, 0])
    h = _ll_block(h, w2b_ref[...], b2b_ref[...], s2b_ref[0, 0])
    o_ref[0] = h + out + scu_ref[0]


@functools.partial(jax.jit, static_argnames=())
def kernel(x, W1a, b1a, s1a, W1b, b1b, s1b, Wg, bg, sg, W2a, b2a, s2a,
           W2b, b2b, s2b):
    B, C, H, W = x.shape
    N = H * W
    f32 = jnp.float32

    def v(a):
        return jnp.asarray(a, f32).reshape(1, -1)

    def sc(a):
        return jnp.asarray(a, f32).reshape(1, 1)

    xcn = x.reshape(B, C, N)

    wspec = pl.BlockSpec((C, C), lambda b: (0, 0))
    bspec = pl.BlockSpec((1, C), lambda b: (0, 0))
    sspec = pl.BlockSpec((1, 1), lambda b: (0, 0))

    ffn1 = pl.pallas_call(
        _ffn_kernel,
        grid=(B,),
        in_specs=[
            pl.BlockSpec((1, C, N), lambda b: (b, 0, 0)),
            wspec, bspec, sspec, wspec, bspec, sspec,
        ],
        out_specs=pl.BlockSpec((1, N, C), lambda b: (b, 0, 0)),
        out_shape=jax.ShapeDtypeStruct((B, N, C), f32),
    )
    y1 = ffn1(xcn, W1a, v(b1a), sc(s1a), W1b, v(b1b), sc(s1b))

    y1v = y1.reshape(B, C, N)
    scu = x.reshape(B, N, C)

    Wg1 = Wg[:, :C]
    Wg2 = Wg[:, C:]

    graph = pl.pallas_call(
        _graph_kernel,
        grid=(B,),
        in_specs=[
            pl.BlockSpec((1, C, N), lambda b: (b, 0, 0)),
            pl.BlockSpec((1, N, C), lambda b: (b, 0, 0)),
            wspec, wspec, bspec, sspec,
            wspec, bspec, sspec, wspec, bspec, sspec,
        ],
        out_specs=pl.BlockSpec((1, N, C), lambda b: (b, 0, 0)),
        out_shape=jax.ShapeDtypeStruct((B, N, C), f32),
    )
    z = graph(y1v, scu, Wg1, Wg2, v(bg), sc(sg),
              W2a, v(b2a), sc(s2a), W2b, v(b2b), sc(s2b))

    return z.reshape(B, C, H, W)
